# baseline (speedup 1.0000x reference)
import functools

import jax
import jax.numpy as jnp
from jax import lax
from jax.experimental import pallas as pl
from jax.experimental.pallas import tpu as pltpu
from jax.experimental.pallas import tpu_sc as plsc

U = 10000
I = 10000
D = 64
L = 2
T = 2
E = 320000
LAMB = 0.5
L2N = 1e-4
EPS = 1e-8
DT = D * (L + 1)

NC = 2
NS = 16
LN = 16
NW = NC * NS

CH = 128
GPT = 80
EP = NC * NS * GPT * CH
SS = 4096
NSTR = EP // NC // SS
NA = 10112
DUMMY = 10000
RPT = NA // NS

BR = 1000
GRID = U // BR

NB = 1024
IPAD = 112
NI = 100
DS = 3 * DT
NV = DS // LN
C2 = (1.0 - LAMB) / float(T)


@functools.lru_cache(maxsize=None)
def _make_segsum(w, n):
    mesh = plsc.VectorSubcoreMesh(core_axis_name="c", subcore_axis_name="s",
                                  num_cores=NC, num_subcores=NS)
    out_type = [jax.ShapeDtypeStruct((NC, NA * w), jnp.float32)
                for _ in range(n)]
    scratch = [
        pltpu.VMEM(((RPT + 8) * w,), jnp.float32),
        pltpu.VMEM((CH, w), jnp.float32),
        pltpu.VMEM((CH, w), jnp.float32),
        pltpu.VMEM((SS // CH, CH), jnp.int32),
        pltpu.VMEM((SS // CH, CH), jnp.int32),
        pltpu.VMEM((SS + CH + LN,), jnp.int32),
        pltpu.VMEM((SS + CH + LN,), jnp.int32),
        pltpu.SemaphoreType.DMA,
        pltpu.SemaphoreType.DMA,
    ]

    @functools.partial(
        pl.kernel, out_type=out_type, mesh=mesh, scratch_types=scratch,
        compiler_params=pltpu.CompilerParams(use_tc_tiling_on_sc=False,
                                             needs_layout_passes=False))
    def seg_kernel(*refs):
        tables = refs[:n]
        srcs = refs[n:2 * n]
        dsts = refs[2 * n:3 * n]
        zflat = refs[3 * n]
        outs = refs[3 * n + 1:3 * n + 1 + n]
        (acc, gb0, gb1, sbuf, dbuf, sfil, dfil,
         sem0, sem1) = refs[3 * n + 1 + n:]
        gbufs = (gb0, gb1)
        sems = (sem0, sem1)

        c = lax.axis_index("c")
        s = lax.axis_index("s")
        lo = s * RPT
        lanes = lax.iota(jnp.int32, LN)
        cbase = c * (EP // (NC * CH))

        zi = jnp.zeros((LN,), jnp.int32)
        dumv = jnp.full((LN,), RPT, jnp.int32)

        def init_body(v, _):
            sfil[pl.ds(v * LN, LN)] = zi
            dfil[pl.ds(v * LN, LN)] = dumv
            return 0
        lax.fori_loop(0, (SS + CH + LN) // LN, init_body, 0)

        for i in range(n):
            pltpu.sync_copy(zflat.at[pl.ds(0, RPT * w)], acc.at[pl.ds(0, RPT * w)])
            pltpu.sync_copy(zflat.at[pl.ds(0, 8 * w)],
                            acc.at[pl.ds(RPT * w, 8 * w)])

            def strip_body(st, _, i=i):
                base = cbase + st * (SS // CH)
                pltpu.sync_copy(srcs[i].at[pl.ds(base, SS // CH)], sbuf)
                pltpu.sync_copy(dsts[i].at[pl.ds(base, SS // CH)], dbuf)

                def filt(v, runvec):
                    r = v >> 3
                    col = (v & 7) << 4
                    dd = dbuf[r, pl.ds(col, LN)]
                    sv = sbuf[r, pl.ds(col, LN)]
                    m = (dd >= lo) & (dd < lo + RPT)
                    mi = m.astype(jnp.int32)
                    pos = plsc.cumsum(mi) - mi + runvec
                    plsc.store_scatter(dfil, [pos], dd - lo, mask=m)
                    plsc.store_scatter(sfil, [pos], sv, mask=m)
                    return runvec + plsc.all_reduce_population_count(m)

                runvec = lax.fori_loop(0, SS // LN, filt, zi)
                cnt = runvec[0]
                ncha = (cnt + CH - 1) >> 7
                for j in range(CH // LN):
                    dfil[pl.ds(cnt + j * LN, LN)] = dumv

                def issue(k2, b, i=i):
                    pltpu.async_copy(
                        tables[i].at[sfil.at[pl.ds(k2 * CH, CH)]],
                        gbufs[b], sems[b])

                @pl.when(ncha > 0)
                def _():
                    issue(jnp.int32(0), 0)

                def chunk(k2, _, i=i):
                    for b in range(2):
                        @pl.when((k2 & 1) == b)
                        def _(b=b):
                            @pl.when(k2 + 1 < ncha)
                            def _():
                                issue(k2 + 1, 1 - b)
                            pltpu.make_async_copy(
                                tables[i].at[sfil.at[pl.ds(k2 * CH, CH)]],
                                gbufs[b], sems[b]).wait()

                            def grp(g, _):
                                e0 = k2 * CH + g * LN
                                dvec = dfil[pl.ds(e0, LN)]
                                obase = dvec * w
                                rowv = g * LN + lanes
                                for p in range(w):
                                    colv = jnp.full((LN,), p, jnp.int32)
                                    vals = plsc.load_gather(
                                        gbufs[b], [rowv, colv])
                                    plsc.addupdate_scatter(
                                        acc, [obase + p], vals)
                                return 0

                            lax.fori_loop(0, CH // LN, grp, 0)
                    return 0

                lax.fori_loop(0, ncha, chunk, 0)
                return 0

            lax.fori_loop(0, NSTR, strip_body, 0)
            pltpu.sync_copy(acc.at[pl.ds(0, RPT * w)],
                            outs[i].at[c, pl.ds(lo * w, RPT * w)])

    return seg_kernel


def _score_body(ubig, ibig, users, items, scores_out, l2_out,
                uidx_v, urows_v, iidx_v, irows_v, sc_v, sem):
    c = lax.axis_index("c")
    s = lax.axis_index("s")
    wid = s * NC + c
    upt = NB // NW
    ubase = wid * upt
    pltpu.sync_copy(users.at[pl.ds(ubase, upt)], uidx_v)
    pltpu.async_copy(ubig.at[uidx_v], urows_v, sem).wait()

    def user_body(ul, l2acc):
        pltpu.sync_copy(items.at[ubase + ul], iidx_v)
        pltpu.async_copy(ibig.at[iidx_v], irows_v, sem).wait()
        urow = [urows_v[ul, pl.ds(k * LN, LN)] for k in range(NV)]
        l2u = jnp.zeros((LN,), jnp.float32)
        for k in range(DT // LN):
            l2u = l2u + urow[k] * urow[k]
        l2acc = l2acc + jnp.sum(l2u) * (float(NI) / (LAMB * LAMB))

        def grp_body(jg, carry):
            l2i = carry
            svec = jnp.zeros((LN,), jnp.float32)
            lanes = lax.iota(jnp.int32, LN)
            for jj in range(LN):
                j = jg * LN + jj
                acc = jnp.zeros((LN,), jnp.float32)
                acc2 = jnp.zeros((LN,), jnp.float32)
                for k in range(NV):
                    r = irows_v[j, pl.ds(k * LN, LN)]
                    acc = acc + urow[k] * r
                    if k < DT // LN:
                        acc2 = acc2 + r * r
                sj = jnp.sum(acc)
                valid = j < NI
                l2i = l2i + jnp.where(valid, jnp.sum(acc2), 0.0)
                svec = jnp.where(lanes == jj, svec + sj, svec)
            sc_v[pl.ds(jg * LN, LN)] = svec
            return l2i

        l2acc = lax.fori_loop(0, IPAD // LN, grp_body, l2acc)
        pltpu.sync_copy(sc_v, scores_out.at[ubase + ul])
        return l2acc

    l2tot = lax.fori_loop(0, upt, user_body, jnp.float32(0.0))
    lanes = lax.iota(jnp.int32, LN)
    sc_v[pl.ds(0, LN)] = jnp.where(lanes == 0, l2tot, 0.0)
    pltpu.sync_copy(sc_v.at[pl.ds(0, LN)], l2_out.at[pl.ds(wid * LN, LN)])


@functools.lru_cache(maxsize=None)
def _make_score():
    return functools.partial(
        pl.kernel,
        out_type=[jax.ShapeDtypeStruct((NB, IPAD), jnp.float32),
                  jax.ShapeDtypeStruct((NW * LN,), jnp.float32)],
        mesh=plsc.VectorSubcoreMesh(core_axis_name="c", subcore_axis_name="s",
                                    num_cores=NC, num_subcores=NS),
        scratch_types=[
            pltpu.VMEM((NB // NW,), jnp.int32),
            pltpu.VMEM((NB // NW, DS), jnp.float32),
            pltpu.VMEM((IPAD,), jnp.int32),
            pltpu.VMEM((IPAD, DS), jnp.float32),
            pltpu.VMEM((IPAD,), jnp.float32),
            pltpu.SemaphoreType.DMA,
        ],
        compiler_params=pltpu.CompilerParams(use_tc_tiling_on_sc=False,
                                             needs_layout_passes=False),
    )(_score_body)


def _full(shape):
    return pl.BlockSpec(shape, lambda i: (0,) * len(shape))


def _blk3(w):
    return pl.BlockSpec((2, BR, w), lambda i: (0, i, 0))


def _blk2(w):
    return pl.BlockSpec((BR, w), lambda i: (i, 0))


def _tc_b_body(pr0, pr1, pt, pi0, pi1, mg, w0, wii,
               u1_o, i1_o, s10_o, s11_o, c0_o, c1_o, aux_o):
    m = mg[...]
    ew = jnp.exp(m - jnp.max(m))
    wv = ew / jnp.sum(ew)
    wa = wv[0:1, 0:1]
    wb = wv[0:1, 1:2]
    pr0b = pr0[0] + pr0[1]
    pr1b = pr1[0] + pr1[1]
    d0 = pr0b[:, D:D + 1]
    d1 = pr1b[:, D:D + 1]
    tw = d0 * wa + d1 * wb
    coef0 = (d0 * wa / (tw + EPS)) / (d0 + EPS)
    coef1 = (d1 * wb / (tw + EPS)) / (d1 + EPS)
    uagg = coef0 * pr0b[:, :D] + coef1 * pr1b[:, :D]
    w0m = w0[...]
    u1 = jnp.dot(uagg, w0m, preferred_element_type=jnp.float32)
    ptb = pt[0] + pt[1]
    idg = ptb[:, D:D + 1] + EPS
    i1 = jnp.dot(ptb[:, :D] / idg, w0m, preferred_element_type=jnp.float32)
    pib0 = pi0[0] + pi0[1]
    pib1 = pi1[0] + pi1[1]
    ii0 = pib0[:, D:D + 1] + EPS
    ii1 = pib1[:, D:D + 1] + EPS
    s10 = jnp.dot(pib0[:, :D] / ii0, wii[0], preferred_element_type=jnp.float32)
    s11 = jnp.dot(pib1[:, :D] / ii1, wii[1], preferred_element_type=jnp.float32)
    u1_o[...] = u1
    i1_o[...] = i1
    s10_o[...] = s10
    s11_o[...] = s11
    c0_o[...] = jnp.concatenate([i1, s10], axis=1)
    c1_o[...] = jnp.concatenate([i1, s11], axis=1)
    aux_o[...] = jnp.concatenate(
        [coef0, coef1, 1.0 / idg, 1.0 / ii0, 1.0 / ii1,
         1.0 / (d0 + EPS), 1.0 / (d1 + EPS), jnp.zeros_like(d0)], axis=1)


def _tc_b(pr0, pr1, pt, pi0, pi1, mg, w0, wii):
    return pl.pallas_call(
        _tc_b_body,
        grid=(GRID,),
        in_specs=[_blk3(80), _blk3(80), _blk3(80), _blk3(80), _blk3(80),
                  _full((1, 2)), _full((D, D)), _full((T, D, D))],
        out_specs=[_blk2(D), _blk2(D), _blk2(D), _blk2(D),
                   _blk2(2 * D), _blk2(2 * D), _blk2(8)],
        out_shape=[jax.ShapeDtypeStruct((U, D), jnp.float32)] * 4
        + [jax.ShapeDtypeStruct((U, 2 * D), jnp.float32)] * 2
        + [jax.ShapeDtypeStruct((U, 8), jnp.float32)],
    )(pr0, pr1, pt, pi0, pi1, mg, w0, wii)


def _tc_d_body(pt2, prl0, prl1, pii0, pii1, aux, w1, wii1,
               u2_o, i2_o, s20_o, s21_o):
    a = aux[...]
    coef0 = a[:, 0:1]
    coef1 = a[:, 1:2]
    invid = a[:, 2:3]
    invii0 = a[:, 3:4]
    invii1 = a[:, 4:5]
    prl0b = prl0[0] + prl0[1]
    prl1b = prl1[0] + prl1[1]
    uagg = coef0 * prl0b + coef1 * prl1b
    w1m = w1[...]
    u2_o[...] = jnp.dot(uagg, w1m, preferred_element_type=jnp.float32)
    pt2b = pt2[0] + pt2[1]
    i2_o[...] = jnp.dot(pt2b * invid, w1m, preferred_element_type=jnp.float32)
    pii0b = pii0[0] + pii0[1]
    pii1b = pii1[0] + pii1[1]
    s20_o[...] = jnp.dot(pii0b * invii0, wii1[0],
                         preferred_element_type=jnp.float32)
    s21_o[...] = jnp.dot(pii1b * invii1, wii1[1],
                         preferred_element_type=jnp.float32)


def _tc_d(pt2, prl0, prl1, pii0, pii1, aux, w1, wii1):
    return pl.pallas_call(
        _tc_d_body,
        grid=(GRID,),
        in_specs=[_blk3(D), _blk3(D), _blk3(D), _blk3(D), _blk3(D),
                  _blk2(8), _full((D, D)), _full((T, D, D))],
        out_specs=[_blk2(D)] * 4,
        out_shape=[jax.ShapeDtypeStruct((U, D), jnp.float32)] * 4,
    )(pt2, prl0, prl1, pii0, pii1, aux, w1, wii1)


def _tc_f_body(ue, u1, u2, ie, i1, i2, s10, s11, s20, s21,
               pa0, pa1, ru0, ru1, pe0, pe1, aux, ws, ub_o, ib_o):
    a = aux[...]
    invud0 = a[:, 5:6]
    invud1 = a[:, 6:7]
    uf = jnp.concatenate([ue[...], u1[...], u2[...]], axis=1)
    agg0 = pa0[0][:, :D] + pa0[1][:, :D]
    agg1 = pa1[0][:, :D] + pa1[1][:, :D]
    uia0 = jnp.concatenate([agg0, ru0[0] + ru0[1], pe0[0] + pe0[1]],
                           axis=1) * invud0
    uia1 = jnp.concatenate([agg1, ru1[0] + ru1[1], pe1[0] + pe1[1]],
                           axis=1) * invud1
    up0 = jnp.dot(uia0, ws[0], preferred_element_type=jnp.float32)
    up1 = jnp.dot(uia1, ws[1], preferred_element_type=jnp.float32)
    ub_o[...] = jnp.concatenate([LAMB * uf, C2 * up0, C2 * up1], axis=1)
    itf = jnp.concatenate([ie[...], i1[...], i2[...]], axis=1)
    sf0 = jnp.concatenate([ie[...], s10[...], s20[...]], axis=1)
    sf1 = jnp.concatenate([ie[...], s11[...], s21[...]], axis=1)
    ip0 = jnp.dot(sf0, ws[0], preferred_element_type=jnp.float32)
    ip1 = jnp.dot(sf1, ws[1], preferred_element_type=jnp.float32)
    ib_o[...] = jnp.concatenate([itf, ip0, ip1], axis=1)


def _tc_f(ue, u1, u2, ie, i1, i2, s10, s11, s20, s21,
          pa0, pa1, ru0, ru1, pe0, pe1, aux, ws):
    return pl.pallas_call(
        _tc_f_body,
        grid=(GRID,),
        in_specs=[_blk2(D)] * 10
        + [_blk3(80), _blk3(80), _blk3(D), _blk3(D), _blk3(D), _blk3(D),
           _blk2(8), _full((T, DT, DT))],
        out_specs=[_blk2(DS), _blk2(DS)],
        out_shape=[jax.ShapeDtypeStruct((U, DS), jnp.float32)] * 2,
    )(ue, u1, u2, ie, i1, i2, s10, s11, s20, s21,
      pa0, pa1, ru0, ru1, pe0, pe1, aux, ws)


def _pad_edges(src, dst):
    pad = EP - E
    srcp = jnp.concatenate([src.astype(jnp.int32), jnp.zeros((pad,), jnp.int32)])
    dstp = jnp.concatenate([dst.astype(jnp.int32),
                            jnp.full((pad,), DUMMY, jnp.int32)])
    return srcp.reshape(EP // CH, CH), dstp.reshape(EP // CH, CH)


def kernel(users, items, user_embedding, item_embedding, user_item_W,
           item_item_W, item_behavior_W_score, mgnn_weight, rel_user_idx,
           rel_item_idx, train_user_idx, train_item_idx, ii_src_idx,
           ii_dst_idx):
    f32 = jnp.float32
    onespad = jnp.concatenate(
        [jnp.ones((I, 1), f32), jnp.zeros((I, 15), f32)], axis=1)
    item_aug = jnp.concatenate([item_embedding, onespad], axis=1)
    user_aug = jnp.concatenate([user_embedding, onespad], axis=1)

    r0s, r0d = _pad_edges(rel_item_idx[0], rel_user_idx[0])
    r1s, r1d = _pad_edges(rel_item_idx[1], rel_user_idx[1])
    tts, ttd = _pad_edges(train_user_idx, train_item_idx)
    i0s, i0d = _pad_edges(ii_src_idx[0], ii_dst_idx[0])
    i1s, i1d = _pad_edges(ii_src_idx[1], ii_dst_idx[1])

    zf = jnp.zeros((RPT * 80,), f32)

    def _sh(o, w):
        return o.reshape(NC, NA, w)[:, :U, :]

    pa0, pa1, ptr, pi0, pi1 = _make_segsum(80, 5)(
        item_aug, item_aug, user_aug, item_aug, item_aug,
        r0s, r1s, tts, i0s, i1s,
        r0d, r1d, ttd, i0d, i1d,
        zf)

    pa0s = _sh(pa0, 80)
    pa1s = _sh(pa1, 80)

    mg2 = mgnn_weight.reshape(1, T).astype(f32)
    u1, i1, s10, s11, c0, c1, aux = _tc_b(
        pa0s, pa1s, _sh(ptr, 80), _sh(pi0, 80), _sh(pi1, 80),
        mg2, user_item_W[0], item_item_W[:, 0])

    pt2, prI0, prS0, prI1, prS1, pii0, pii1 = _make_segsum(64, 7)(
        u1, i1, s10, i1, s11, s10, s11,
        tts, r0s, r0s, r1s, r1s, i0s, i1s,
        ttd, r0d, r0d, r1d, r1d, i0d, i1d,
        zf)

    u2, i2, s20, s21 = _tc_d(
        _sh(pt2, 64), _sh(prI0, 64), _sh(prI1, 64),
        _sh(pii0, 64), _sh(pii1, 64), aux,
        user_item_W[1], item_item_W[:, 1])

    pe0, pe1 = _make_segsum(64, 2)(s20, s21, r0s, r1s, r0d, r1d, zf)

    ubig, ibig = _tc_f(
        user_embedding, u1, u2, item_embedding, i1, i2, s10, s11, s20, s21,
        pa0s, pa1s, _sh(prS0, 64), _sh(prS1, 64),
        _sh(pe0, 64), _sh(pe1, 64), aux, item_behavior_W_score)

    items_p = jnp.concatenate(
        [items.astype(jnp.int32),
         jnp.zeros((NB, IPAD - NI), jnp.int32)], axis=1)
    scores_p, l2p = _make_score()(ubig, ibig, users.astype(jnp.int32), items_p)

    scores = scores_p[:, :NI]
    l2 = L2N * jnp.sum(l2p)
    return (scores, l2)


# speedup vs baseline: 5.7534x; 5.7534x over previous
import functools

import jax
import jax.numpy as jnp
from jax import lax
from jax.experimental import pallas as pl
from jax.experimental.pallas import tpu as pltpu
from jax.experimental.pallas import tpu_sc as plsc

U = 10000
I = 10000
D = 64
L = 2
T = 2
E = 320000
LAMB = 0.5
L2N = 1e-4
EPS = 1e-8
DT = D * (L + 1)

NC = 2
NS = 16
LN = 16
NW = NC * NS

CH = 128
GPT = 80
EP = NC * NS * GPT * CH
NBUF = 4
NA = 10112
DUMMY = 10000
RPT = NA // NS

BR = 1000
GRID = U // BR

NB = 1024
IPAD = 112
NI = 100
DS = 3 * DT
NV = DS // LN
C2 = (1.0 - LAMB) / float(T)


@functools.lru_cache(maxsize=None)
def _make_segsum(widths):
    n = len(widths)
    dws = sorted(set(widths))
    mesh = plsc.VectorSubcoreMesh(core_axis_name="c", subcore_axis_name="s",
                                  num_cores=NC, num_subcores=NS)
    out_type = [jax.ShapeDtypeStruct((NC, NA, w), jnp.float32) for w in widths]
    scratch = []
    for w in dws:
        scratch.append(pltpu.VMEM_SHARED((NA, w), jnp.float32))
    for w in dws:
        for _ in range(NBUF):
            scratch.append(pltpu.VMEM((CH, w), jnp.float32))
    scratch += [
        pltpu.VMEM((GPT, CH), jnp.int32),
        pltpu.VMEM((GPT, CH), jnp.int32),
    ] + [pltpu.SemaphoreType.DMA] * NBUF

    @functools.partial(
        pl.kernel, out_type=out_type, mesh=mesh, scratch_types=scratch,
        compiler_params=pltpu.CompilerParams(use_tc_tiling_on_sc=False))
    def seg_kernel(*refs):
        tables = refs[:n]
        srcs = refs[n:2 * n]
        dsts = refs[2 * n:3 * n]
        zeros = {w: refs[3 * n + j] for j, w in enumerate(dws)}
        outs = refs[3 * n + len(dws):3 * n + len(dws) + n]
        k = 3 * n + len(dws) + n
        accs = {w: refs[k + j] for j, w in enumerate(dws)}
        k += len(dws)
        rows = {w: [refs[k + j * NBUF + b] for b in range(NBUF)]
                for j, w in enumerate(dws)}
        k += len(dws) * NBUF
        sidx = refs[k]
        didx = refs[k + 1]
        sems = refs[k + 2:k + 2 + NBUF]

        c = lax.axis_index("c")
        s = lax.axis_index("s")
        rowbase = c * (EP // (NC * CH)) + s * GPT
        for i in range(n):
            w = widths[i]
            acc = accs[w]
            ring = rows[w]
            pltpu.sync_copy(zeros[w].at[pl.ds(s * RPT, RPT)],
                            acc.at[pl.ds(s * RPT, RPT)])
            pltpu.sync_copy(srcs[i].at[pl.ds(rowbase, GPT)], sidx)
            pltpu.sync_copy(dsts[i].at[pl.ds(rowbase, GPT)], didx)
            plsc.subcore_barrier()
            for b in range(NBUF):
                pltpu.async_copy(tables[i].at[sidx.at[b]], ring[b], sems[b])

            def body(gg, _, i=i, ring=ring, acc=acc):
                for b in range(NBUF):
                    g = gg * NBUF + b
                    pltpu.make_async_copy(tables[i].at[sidx.at[g]],
                                          ring[b], sems[b]).wait()
                    pltpu.sync_copy(ring[b], acc.at[didx.at[g]], add=True)
                    nxt = g + NBUF

                    @pl.when(nxt < GPT)
                    def _():
                        pltpu.async_copy(tables[i].at[sidx.at[nxt]],
                                         ring[b], sems[b])
                return 0

            lax.fori_loop(0, GPT // NBUF, body, 0)
            plsc.subcore_barrier()
            pltpu.sync_copy(acc.at[pl.ds(s * RPT, RPT)],
                            outs[i].at[c, pl.ds(s * RPT, RPT)])
            plsc.subcore_barrier()

    return seg_kernel


SPA = 64
SPB = IPAD - SPA


def _score_body(ubig, ibig, users, items, scores_out, l2_out,
                uidx_v, urows_v, iidx2_v, ira_v, irb_v, scb_v,
                semu, sema, semb):
    c = lax.axis_index("c")
    s = lax.axis_index("s")
    wid = s * NC + c
    upt = NB // NW
    ubase = wid * upt
    pltpu.sync_copy(users.at[pl.ds(ubase, upt)], uidx_v)
    pltpu.sync_copy(items.at[pl.ds(ubase, upt)], iidx2_v)
    pltpu.async_copy(ubig.at[uidx_v], urows_v, semu).wait()

    def ga(ul):
        return pltpu.make_async_copy(
            ibig.at[iidx2_v.at[ul, pl.ds(0, SPA)]], ira_v, sema)

    def gb(ul):
        return pltpu.make_async_copy(
            ibig.at[iidx2_v.at[ul, pl.ds(SPA, SPB)]], irb_v, semb)

    ga(jnp.int32(0)).start()
    gb(jnp.int32(0)).start()

    def user_body(ul, l2acc):
        urow = [urows_v[ul, pl.ds(k * LN, LN)] for k in range(NV)]
        l2u = jnp.zeros((LN,), jnp.float32)
        for k in range(DT // LN):
            l2u = l2u + urow[k] * urow[k]
        l2acc = l2acc + jnp.sum(l2u) * (float(NI) / (LAMB * LAMB))
        lanes = lax.iota(jnp.int32, LN)

        def mk_grp(buf, joff):
            def grp_body(jg, carry):
                l2i = carry
                svec = jnp.zeros((LN,), jnp.float32)
                for jj in range(LN):
                    j = jg * LN + jj
                    acc = jnp.zeros((LN,), jnp.float32)
                    acc2 = jnp.zeros((LN,), jnp.float32)
                    for k in range(NV):
                        r = buf[j - joff, pl.ds(k * LN, LN)]
                        acc = acc + urow[k] * r
                        if k < DT // LN:
                            acc2 = acc2 + r * r
                    sj = jnp.sum(acc)
                    valid = j < NI
                    l2i = l2i + jnp.where(valid, jnp.sum(acc2), 0.0)
                    svec = jnp.where(lanes == jj, svec + sj, svec)
                scb_v[ul, pl.ds(jg * LN, LN)] = svec
                return l2i
            return grp_body

        ga(ul).wait()
        l2acc = lax.fori_loop(0, SPA // LN, mk_grp(ira_v, 0), l2acc)

        @pl.when(ul + 1 < upt)
        def _():
            ga(ul + 1).start()

        gb(ul).wait()
        l2acc = lax.fori_loop(SPA // LN, IPAD // LN, mk_grp(irb_v, SPA), l2acc)

        @pl.when(ul + 1 < upt)
        def _():
            gb(ul + 1).start()
        return l2acc

    l2tot = lax.fori_loop(0, upt, user_body, jnp.float32(0.0))
    pltpu.sync_copy(scb_v, scores_out.at[pl.ds(ubase, upt)])
    lanes = lax.iota(jnp.int32, LN)
    scb_v[0, pl.ds(0, LN)] = jnp.where(lanes == 0, l2tot, 0.0)
    pltpu.sync_copy(scb_v.at[0, pl.ds(0, LN)], l2_out.at[pl.ds(wid * LN, LN)])


@functools.lru_cache(maxsize=None)
def _make_score():
    return functools.partial(
        pl.kernel,
        out_type=[jax.ShapeDtypeStruct((NB, IPAD), jnp.float32),
                  jax.ShapeDtypeStruct((NW * LN,), jnp.float32)],
        mesh=plsc.VectorSubcoreMesh(core_axis_name="c", subcore_axis_name="s",
                                    num_cores=NC, num_subcores=NS),
        scratch_types=[
            pltpu.VMEM((NB // NW,), jnp.int32),
            pltpu.VMEM((NB // NW, DS), jnp.float32),
            pltpu.VMEM((NB // NW, IPAD), jnp.int32),
            pltpu.VMEM((SPA, DS), jnp.float32),
            pltpu.VMEM((SPB, DS), jnp.float32),
            pltpu.VMEM((NB // NW, IPAD), jnp.float32),
            pltpu.SemaphoreType.DMA,
            pltpu.SemaphoreType.DMA,
            pltpu.SemaphoreType.DMA,
        ],
        compiler_params=pltpu.CompilerParams(use_tc_tiling_on_sc=False,
                                             needs_layout_passes=False),
    )(_score_body)


def _full(shape):
    return pl.BlockSpec(shape, lambda i: (0,) * len(shape))


def _blk3(w):
    return pl.BlockSpec((2, BR, w), lambda i: (0, i, 0))


def _blk2(w):
    return pl.BlockSpec((BR, w), lambda i: (i, 0))


def _tc_b_body(pr0, pr1, pt, pi0, pi1, mg, w0, wii,
               u1_o, i1_o, s10_o, s11_o, c0_o, c1_o, aux_o):
    m = mg[...]
    ew = jnp.exp(m - jnp.max(m))
    wv = ew / jnp.sum(ew)
    wa = wv[0:1, 0:1]
    wb = wv[0:1, 1:2]
    pr0b = pr0[0] + pr0[1]
    pr1b = pr1[0] + pr1[1]
    d0 = pr0b[:, D:D + 1]
    d1 = pr1b[:, D:D + 1]
    tw = d0 * wa + d1 * wb
    coef0 = (d0 * wa / (tw + EPS)) / (d0 + EPS)
    coef1 = (d1 * wb / (tw + EPS)) / (d1 + EPS)
    uagg = coef0 * pr0b[:, :D] + coef1 * pr1b[:, :D]
    w0m = w0[...]
    u1 = jnp.dot(uagg, w0m, preferred_element_type=jnp.float32)
    ptb = pt[0] + pt[1]
    idg = ptb[:, D:D + 1] + EPS
    i1 = jnp.dot(ptb[:, :D] / idg, w0m, preferred_element_type=jnp.float32)
    pib0 = pi0[0] + pi0[1]
    pib1 = pi1[0] + pi1[1]
    ii0 = pib0[:, D:D + 1] + EPS
    ii1 = pib1[:, D:D + 1] + EPS
    s10 = jnp.dot(pib0[:, :D] / ii0, wii[0], preferred_element_type=jnp.float32)
    s11 = jnp.dot(pib1[:, :D] / ii1, wii[1], preferred_element_type=jnp.float32)
    u1_o[...] = u1
    i1_o[...] = i1
    s10_o[...] = s10
    s11_o[...] = s11
    c0_o[...] = jnp.concatenate([i1, s10], axis=1)
    c1_o[...] = jnp.concatenate([i1, s11], axis=1)
    aux_o[...] = jnp.concatenate(
        [coef0, coef1, 1.0 / idg, 1.0 / ii0, 1.0 / ii1,
         1.0 / (d0 + EPS), 1.0 / (d1 + EPS), jnp.zeros_like(d0)], axis=1)


def _tc_b(pr0, pr1, pt, pi0, pi1, mg, w0, wii):
    return pl.pallas_call(
        _tc_b_body,
        grid=(GRID,),
        in_specs=[_blk3(80), _blk3(80), _blk3(80), _blk3(80), _blk3(80),
                  _full((1, 2)), _full((D, D)), _full((T, D, D))],
        out_specs=[_blk2(D), _blk2(D), _blk2(D), _blk2(D),
                   _blk2(2 * D), _blk2(2 * D), _blk2(8)],
        out_shape=[jax.ShapeDtypeStruct((U, D), jnp.float32)] * 4
        + [jax.ShapeDtypeStruct((U, 2 * D), jnp.float32)] * 2
        + [jax.ShapeDtypeStruct((U, 8), jnp.float32)],
    )(pr0, pr1, pt, pi0, pi1, mg, w0, wii)


def _tc_d_body(pt2, prl0, prl1, pii0, pii1, aux, w1, wii1,
               u2_o, i2_o, s20_o, s21_o):
    a = aux[...]
    coef0 = a[:, 0:1]
    coef1 = a[:, 1:2]
    invid = a[:, 2:3]
    invii0 = a[:, 3:4]
    invii1 = a[:, 4:5]
    prl0b = prl0[0] + prl0[1]
    prl1b = prl1[0] + prl1[1]
    uagg = coef0 * prl0b + coef1 * prl1b
    w1m = w1[...]
    u2_o[...] = jnp.dot(uagg, w1m, preferred_element_type=jnp.float32)
    pt2b = pt2[0] + pt2[1]
    i2_o[...] = jnp.dot(pt2b * invid, w1m, preferred_element_type=jnp.float32)
    pii0b = pii0[0] + pii0[1]
    pii1b = pii1[0] + pii1[1]
    s20_o[...] = jnp.dot(pii0b * invii0, wii1[0],
                         preferred_element_type=jnp.float32)
    s21_o[...] = jnp.dot(pii1b * invii1, wii1[1],
                         preferred_element_type=jnp.float32)


def _tc_d(pt2, prl0, prl1, pii0, pii1, aux, w1, wii1):
    return pl.pallas_call(
        _tc_d_body,
        grid=(GRID,),
        in_specs=[_blk3(D), _blk3(D), _blk3(D), _blk3(D), _blk3(D),
                  _blk2(8), _full((D, D)), _full((T, D, D))],
        out_specs=[_blk2(D)] * 4,
        out_shape=[jax.ShapeDtypeStruct((U, D), jnp.float32)] * 4,
    )(pt2, prl0, prl1, pii0, pii1, aux, w1, wii1)


def _tc_f_body(ue, u1, u2, ie, i1, i2, s10, s11, s20, s21,
               pa0, pa1, ru0, ru1, pe0, pe1, aux, ws, ub_o, ib_o):
    a = aux[...]
    invud0 = a[:, 5:6]
    invud1 = a[:, 6:7]
    uf = jnp.concatenate([ue[...], u1[...], u2[...]], axis=1)
    agg0 = pa0[0][:, :D] + pa0[1][:, :D]
    agg1 = pa1[0][:, :D] + pa1[1][:, :D]
    uia0 = jnp.concatenate([agg0, ru0[0] + ru0[1], pe0[0] + pe0[1]],
                           axis=1) * invud0
    uia1 = jnp.concatenate([agg1, ru1[0] + ru1[1], pe1[0] + pe1[1]],
                           axis=1) * invud1
    up0 = jnp.dot(uia0, ws[0], preferred_element_type=jnp.float32)
    up1 = jnp.dot(uia1, ws[1], preferred_element_type=jnp.float32)
    ub_o[...] = jnp.concatenate([LAMB * uf, C2 * up0, C2 * up1], axis=1)
    itf = jnp.concatenate([ie[...], i1[...], i2[...]], axis=1)
    sf0 = jnp.concatenate([ie[...], s10[...], s20[...]], axis=1)
    sf1 = jnp.concatenate([ie[...], s11[...], s21[...]], axis=1)
    ip0 = jnp.dot(sf0, ws[0], preferred_element_type=jnp.float32)
    ip1 = jnp.dot(sf1, ws[1], preferred_element_type=jnp.float32)
    ib_o[...] = jnp.concatenate([itf, ip0, ip1], axis=1)


def _tc_f(ue, u1, u2, ie, i1, i2, s10, s11, s20, s21,
          pa0, pa1, ru0, ru1, pe0, pe1, aux, ws):
    return pl.pallas_call(
        _tc_f_body,
        grid=(GRID,),
        in_specs=[_blk2(D)] * 10
        + [_blk3(80), _blk3(80), _blk3(D), _blk3(D), _blk3(D), _blk3(D),
           _blk2(8), _full((T, DT, DT))],
        out_specs=[_blk2(DS), _blk2(DS)],
        out_shape=[jax.ShapeDtypeStruct((U, DS), jnp.float32)] * 2,
    )(ue, u1, u2, ie, i1, i2, s10, s11, s20, s21,
      pa0, pa1, ru0, ru1, pe0, pe1, aux, ws)


def _pad_edges(src, dst):
    pad = EP - E
    srcp = jnp.concatenate([src.astype(jnp.int32), jnp.zeros((pad,), jnp.int32)])
    dstp = jnp.concatenate([dst.astype(jnp.int32),
                            jnp.full((pad,), DUMMY, jnp.int32)])
    return srcp.reshape(EP // CH, CH), dstp.reshape(EP // CH, CH)


def kernel(users, items, user_embedding, item_embedding, user_item_W,
           item_item_W, item_behavior_W_score, mgnn_weight, rel_user_idx,
           rel_item_idx, train_user_idx, train_item_idx, ii_src_idx,
           ii_dst_idx):
    f32 = jnp.float32
    onespad = jnp.concatenate(
        [jnp.ones((I, 1), f32), jnp.zeros((I, 15), f32)], axis=1)
    item_aug = jnp.concatenate([item_embedding, onespad], axis=1)
    user_aug = jnp.concatenate([user_embedding, onespad], axis=1)

    r0s, r0d = _pad_edges(rel_item_idx[0], rel_user_idx[0])
    r1s, r1d = _pad_edges(rel_item_idx[1], rel_user_idx[1])
    tts, ttd = _pad_edges(train_user_idx, train_item_idx)
    i0s, i0d = _pad_edges(ii_src_idx[0], ii_dst_idx[0])
    i1s, i1d = _pad_edges(ii_src_idx[1], ii_dst_idx[1])

    z64 = jnp.zeros((NA, 64), f32)
    z80 = jnp.zeros((NA, 80), f32)

    pa0, pa1, ptr, pi0, pi1 = _make_segsum((80, 80, 80, 80, 80))(
        item_aug, item_aug, user_aug, item_aug, item_aug,
        r0s, r1s, tts, i0s, i1s,
        r0d, r1d, ttd, i0d, i1d,
        z80)

    pa0s = pa0[:, :U, :]
    pa1s = pa1[:, :U, :]

    mg2 = mgnn_weight.reshape(1, T).astype(f32)
    u1, i1, s10, s11, c0, c1, aux = _tc_b(
        pa0s, pa1s, ptr[:, :U, :], pi0[:, :U, :], pi1[:, :U, :],
        mg2, user_item_W[0], item_item_W[:, 0])

    pt2, prI0, prS0, prI1, prS1, pii0, pii1 = _make_segsum((64,) * 7)(
        u1, i1, s10, i1, s11, s10, s11,
        tts, r0s, r0s, r1s, r1s, i0s, i1s,
        ttd, r0d, r0d, r1d, r1d, i0d, i1d,
        z64)

    u2, i2, s20, s21 = _tc_d(
        pt2[:, :U, :], prI0[:, :U, :], prI1[:, :U, :],
        pii0[:, :U, :], pii1[:, :U, :], aux,
        user_item_W[1], item_item_W[:, 1])

    pe0, pe1 = _make_segsum((64, 64))(s20, s21, r0s, r1s, r0d, r1d, z64)

    ubig, ibig = _tc_f(
        user_embedding, u1, u2, item_embedding, i1, i2, s10, s11, s20, s21,
        pa0s, pa1s, prS0[:, :U, :], prS1[:, :U, :],
        pe0[:, :U, :], pe1[:, :U, :], aux, item_behavior_W_score)

    items_p = jnp.concatenate(
        [items.astype(jnp.int32),
         jnp.zeros((NB, IPAD - NI), jnp.int32)], axis=1)
    scores_p, l2p = _make_score()(ubig, ibig, users.astype(jnp.int32), items_p)

    scores = scores_p[:, :NI]
    l2 = L2N * jnp.sum(l2p)
    return (scores, l2)


# speedup vs baseline: 5.8188x; 1.0114x over previous
import functools

import jax
import jax.numpy as jnp
from jax import lax
from jax.experimental import pallas as pl
from jax.experimental.pallas import tpu as pltpu
from jax.experimental.pallas import tpu_sc as plsc

U = 10000
I = 10000
D = 64
L = 2
T = 2
E = 320000
LAMB = 0.5
L2N = 1e-4
EPS = 1e-8
DT = D * (L + 1)

NC = 2
NS = 16
LN = 16
NW = NC * NS

CH = 128
GPT = 80
EP = NC * NS * GPT * CH
NBUF = 4
NA = 10112
DUMMY = 10000
RPT = NA // NS

BR = 1000
GRID = U // BR

NB = 1024
IPAD = 112
NI = 100
DSI = 7 * D
DSU = DSI + LN
NV = DSI // LN
C2 = (1.0 - LAMB) / float(T)


@functools.lru_cache(maxsize=None)
def _make_segsum(widths):
    n = len(widths)
    dws = sorted(set(widths))
    mesh = plsc.VectorSubcoreMesh(core_axis_name="c", subcore_axis_name="s",
                                  num_cores=NC, num_subcores=NS)
    out_type = [jax.ShapeDtypeStruct((NC, NA, w), jnp.float32) for w in widths]
    scratch = []
    for w in dws:
        scratch.append(pltpu.VMEM_SHARED((NA, w), jnp.float32))
    for w in dws:
        for _ in range(NBUF):
            scratch.append(pltpu.VMEM((CH, w), jnp.float32))
    scratch += [
        pltpu.VMEM((GPT, CH), jnp.int32),
        pltpu.VMEM((GPT, CH), jnp.int32),
    ] + [pltpu.SemaphoreType.DMA] * NBUF

    @functools.partial(
        pl.kernel, out_type=out_type, mesh=mesh, scratch_types=scratch,
        compiler_params=pltpu.CompilerParams(use_tc_tiling_on_sc=False))
    def seg_kernel(*refs):
        tables = refs[:n]
        srcs = refs[n:2 * n]
        dsts = refs[2 * n:3 * n]
        zeros = {w: refs[3 * n + j] for j, w in enumerate(dws)}
        outs = refs[3 * n + len(dws):3 * n + len(dws) + n]
        k = 3 * n + len(dws) + n
        accs = {w: refs[k + j] for j, w in enumerate(dws)}
        k += len(dws)
        rows = {w: [refs[k + j * NBUF + b] for b in range(NBUF)]
                for j, w in enumerate(dws)}
        k += len(dws) * NBUF
        sidx = refs[k]
        didx = refs[k + 1]
        sems = refs[k + 2:k + 2 + NBUF]

        c = lax.axis_index("c")
        s = lax.axis_index("s")
        rowbase = c * (EP // (NC * CH)) + s * GPT
        for i in range(n):
            w = widths[i]
            acc = accs[w]
            ring = rows[w]
            pltpu.sync_copy(zeros[w].at[pl.ds(s * RPT, RPT)],
                            acc.at[pl.ds(s * RPT, RPT)])
            pltpu.sync_copy(srcs[i].at[pl.ds(rowbase, GPT)], sidx)
            pltpu.sync_copy(dsts[i].at[pl.ds(rowbase, GPT)], didx)
            plsc.subcore_barrier()
            for b in range(NBUF):
                pltpu.async_copy(tables[i].at[sidx.at[b]], ring[b], sems[b])

            def body(gg, _, i=i, ring=ring, acc=acc):
                for b in range(NBUF):
                    g = gg * NBUF + b
                    pltpu.make_async_copy(tables[i].at[sidx.at[g]],
                                          ring[b], sems[b]).wait()
                    pltpu.sync_copy(ring[b], acc.at[didx.at[g]], add=True)
                    nxt = g + NBUF

                    @pl.when(nxt < GPT)
                    def _():
                        pltpu.async_copy(tables[i].at[sidx.at[nxt]],
                                         ring[b], sems[b])
                return 0

            lax.fori_loop(0, GPT // NBUF, body, 0)
            plsc.subcore_barrier()
            pltpu.sync_copy(acc.at[pl.ds(s * RPT, RPT)],
                            outs[i].at[c, pl.ds(s * RPT, RPT)])
            plsc.subcore_barrier()

    return seg_kernel


SPA = 64
SPB = IPAD - SPA


def _score_body(ubig, ibig, users, items, scores_out, l2_out,
                uidx_v, urows_v, iidx2_v, ira_v, irb_v, scb_v,
                semu, sema, semb):
    c = lax.axis_index("c")
    s = lax.axis_index("s")
    wid = s * NC + c
    upt = NB // NW
    ubase = wid * upt
    pltpu.sync_copy(users.at[pl.ds(ubase, upt)], uidx_v)
    pltpu.sync_copy(items.at[pl.ds(ubase, upt)], iidx2_v)
    pltpu.async_copy(ubig.at[uidx_v], urows_v, semu).wait()

    def ga(ul):
        return pltpu.make_async_copy(
            ibig.at[iidx2_v.at[ul, pl.ds(0, SPA)]], ira_v, sema)

    def gb(ul):
        return pltpu.make_async_copy(
            ibig.at[iidx2_v.at[ul, pl.ds(SPA, SPB)]], irb_v, semb)

    ga(jnp.int32(0)).start()
    gb(jnp.int32(0)).start()

    def user_body(ul, l2acc):
        urow = [urows_v[ul, pl.ds(k * LN, LN)] for k in range(NV)]
        ssqv = urows_v[ul, pl.ds(DSI, LN)]
        l2acc = l2acc + ssqv[0] * float(NI)
        lanes = lax.iota(jnp.int32, LN)

        def mk_grp(buf, joff):
            def grp_body(jg, carry):
                l2i = carry
                svec = jnp.zeros((LN,), jnp.float32)
                for jj in range(LN):
                    j = jg * LN + jj
                    acc = jnp.zeros((LN,), jnp.float32)
                    acc2 = jnp.zeros((LN,), jnp.float32)
                    for k in range(NV):
                        r = buf[j - joff, pl.ds(k * LN, LN)]
                        acc = acc + urow[k] * r
                        if k < DT // LN:
                            acc2 = acc2 + r * r
                    sj = jnp.sum(acc)
                    valid = j < NI
                    l2i = l2i + jnp.where(valid, jnp.sum(acc2), 0.0)
                    svec = jnp.where(lanes == jj, svec + sj, svec)
                scb_v[ul, pl.ds(jg * LN, LN)] = svec
                return l2i
            return grp_body

        ga(ul).wait()
        l2acc = lax.fori_loop(0, SPA // LN, mk_grp(ira_v, 0), l2acc)

        @pl.when(ul + 1 < upt)
        def _():
            ga(ul + 1).start()

        gb(ul).wait()
        l2acc = lax.fori_loop(SPA // LN, IPAD // LN, mk_grp(irb_v, SPA), l2acc)

        @pl.when(ul + 1 < upt)
        def _():
            gb(ul + 1).start()
        return l2acc

    l2tot = lax.fori_loop(0, upt, user_body, jnp.float32(0.0))
    pltpu.sync_copy(scb_v, scores_out.at[pl.ds(ubase, upt)])
    lanes = lax.iota(jnp.int32, LN)
    scb_v[0, pl.ds(0, LN)] = jnp.where(lanes == 0, l2tot, 0.0)
    pltpu.sync_copy(scb_v.at[0, pl.ds(0, LN)], l2_out.at[pl.ds(wid * LN, LN)])


@functools.lru_cache(maxsize=None)
def _make_score():
    return functools.partial(
        pl.kernel,
        out_type=[jax.ShapeDtypeStruct((NB, IPAD), jnp.float32),
                  jax.ShapeDtypeStruct((NW * LN,), jnp.float32)],
        mesh=plsc.VectorSubcoreMesh(core_axis_name="c", subcore_axis_name="s",
                                    num_cores=NC, num_subcores=NS),
        scratch_types=[
            pltpu.VMEM((NB // NW,), jnp.int32),
            pltpu.VMEM((NB // NW, DSU), jnp.float32),
            pltpu.VMEM((NB // NW, IPAD), jnp.int32),
            pltpu.VMEM((SPA, DSI), jnp.float32),
            pltpu.VMEM((SPB, DSI), jnp.float32),
            pltpu.VMEM((NB // NW, IPAD), jnp.float32),
            pltpu.SemaphoreType.DMA,
            pltpu.SemaphoreType.DMA,
            pltpu.SemaphoreType.DMA,
        ],
        compiler_params=pltpu.CompilerParams(use_tc_tiling_on_sc=False,
                                             needs_layout_passes=False),
    )(_score_body)


def _full(shape):
    return pl.BlockSpec(shape, lambda i: (0,) * len(shape))


def _blk3(w):
    return pl.BlockSpec((2, BR, w), lambda i: (0, i, 0))


def _blk2(w):
    return pl.BlockSpec((BR, w), lambda i: (i, 0))


def _tc_b_body(pr0, pr1, pt, pi0, pi1, mg, w0, wii,
               u1_o, i1_o, s10_o, s11_o, c0_o, c1_o, aux_o):
    m = mg[...]
    ew = jnp.exp(m - jnp.max(m))
    wv = ew / jnp.sum(ew)
    wa = wv[0:1, 0:1]
    wb = wv[0:1, 1:2]
    pr0b = pr0[0] + pr0[1]
    pr1b = pr1[0] + pr1[1]
    d0 = pr0b[:, D:D + 1]
    d1 = pr1b[:, D:D + 1]
    tw = d0 * wa + d1 * wb
    coef0 = (d0 * wa / (tw + EPS)) / (d0 + EPS)
    coef1 = (d1 * wb / (tw + EPS)) / (d1 + EPS)
    uagg = coef0 * pr0b[:, :D] + coef1 * pr1b[:, :D]
    w0m = w0[...]
    u1 = jnp.dot(uagg, w0m, preferred_element_type=jnp.float32)
    ptb = pt[0] + pt[1]
    idg = ptb[:, D:D + 1] + EPS
    i1 = jnp.dot(ptb[:, :D] / idg, w0m, preferred_element_type=jnp.float32)
    pib0 = pi0[0] + pi0[1]
    pib1 = pi1[0] + pi1[1]
    ii0 = pib0[:, D:D + 1] + EPS
    ii1 = pib1[:, D:D + 1] + EPS
    s10 = jnp.dot(pib0[:, :D] / ii0, wii[0], preferred_element_type=jnp.float32)
    s11 = jnp.dot(pib1[:, :D] / ii1, wii[1], preferred_element_type=jnp.float32)
    u1_o[...] = u1
    i1_o[...] = i1
    s10_o[...] = s10
    s11_o[...] = s11
    c0_o[...] = jnp.concatenate([i1, s10], axis=1)
    c1_o[...] = jnp.concatenate([i1, s11], axis=1)
    aux_o[...] = jnp.concatenate(
        [coef0, coef1, 1.0 / idg, 1.0 / ii0, 1.0 / ii1,
         1.0 / (d0 + EPS), 1.0 / (d1 + EPS), jnp.zeros_like(d0)], axis=1)


def _tc_b(pr0, pr1, pt, pi0, pi1, mg, w0, wii):
    return pl.pallas_call(
        _tc_b_body,
        grid=(GRID,),
        in_specs=[_blk3(80), _blk3(80), _blk3(80), _blk3(80), _blk3(80),
                  _full((1, 2)), _full((D, D)), _full((T, D, D))],
        out_specs=[_blk2(D), _blk2(D), _blk2(D), _blk2(D),
                   _blk2(2 * D), _blk2(2 * D), _blk2(8)],
        out_shape=[jax.ShapeDtypeStruct((U, D), jnp.float32)] * 4
        + [jax.ShapeDtypeStruct((U, 2 * D), jnp.float32)] * 2
        + [jax.ShapeDtypeStruct((U, 8), jnp.float32)],
    )(pr0, pr1, pt, pi0, pi1, mg, w0, wii)


def _tc_d_body(pt2, prl0, prl1, pii0, pii1, aux, w1, wii1,
               u2_o, i2_o, s20_o, s21_o):
    a = aux[...]
    coef0 = a[:, 0:1]
    coef1 = a[:, 1:2]
    invid = a[:, 2:3]
    invii0 = a[:, 3:4]
    invii1 = a[:, 4:5]
    prl0b = prl0[0] + prl0[1]
    prl1b = prl1[0] + prl1[1]
    uagg = coef0 * prl0b + coef1 * prl1b
    w1m = w1[...]
    u2_o[...] = jnp.dot(uagg, w1m, preferred_element_type=jnp.float32)
    pt2b = pt2[0] + pt2[1]
    i2_o[...] = jnp.dot(pt2b * invid, w1m, preferred_element_type=jnp.float32)
    pii0b = pii0[0] + pii0[1]
    pii1b = pii1[0] + pii1[1]
    s20_o[...] = jnp.dot(pii0b * invii0, wii1[0],
                         preferred_element_type=jnp.float32)
    s21_o[...] = jnp.dot(pii1b * invii1, wii1[1],
                         preferred_element_type=jnp.float32)


def _tc_d(pt2, prl0, prl1, pii0, pii1, aux, w1, wii1):
    return pl.pallas_call(
        _tc_d_body,
        grid=(GRID,),
        in_specs=[_blk3(D), _blk3(D), _blk3(D), _blk3(D), _blk3(D),
                  _blk2(8), _full((D, D)), _full((T, D, D))],
        out_specs=[_blk2(D)] * 4,
        out_shape=[jax.ShapeDtypeStruct((U, D), jnp.float32)] * 4,
    )(pt2, prl0, prl1, pii0, pii1, aux, w1, wii1)


def _tc_f_body(ue, u1, u2, ie, i1, i2, s10, s11, s20, s21,
               pa0, pa1, ru0, ru1, pe0, pe1, aux, ws, ub_o, ib_o):
    a = aux[...]
    invud0 = a[:, 5:6]
    invud1 = a[:, 6:7]
    uf = jnp.concatenate([ue[...], u1[...], u2[...]], axis=1)
    agg0 = pa0[0][:, :D] + pa0[1][:, :D]
    agg1 = pa1[0][:, :D] + pa1[1][:, :D]
    uia0 = jnp.concatenate([agg0, ru0[0] + ru0[1], pe0[0] + pe0[1]],
                           axis=1) * invud0
    uia1 = jnp.concatenate([agg1, ru1[0] + ru1[1], pe1[0] + pe1[1]],
                           axis=1) * invud1
    up0 = jnp.dot(jnp.dot(uia0, ws[0], preferred_element_type=jnp.float32),
                  ws[0].T, preferred_element_type=jnp.float32)
    up1 = jnp.dot(jnp.dot(uia1, ws[1], preferred_element_type=jnp.float32),
                  ws[1].T, preferred_element_type=jnp.float32)
    ssq = jnp.sum(uf * uf, axis=1, keepdims=True)
    ub_o[...] = jnp.concatenate(
        [LAMB * ue[...] + C2 * (up0[:, :D] + up1[:, :D]),
         LAMB * u1[...], LAMB * u2[...],
         C2 * up0[:, D:2 * D], C2 * up0[:, 2 * D:3 * D],
         C2 * up1[:, D:2 * D], C2 * up1[:, 2 * D:3 * D],
         ssq, jnp.zeros_like(up0[:, :LN - 1])], axis=1)
    ib_o[...] = jnp.concatenate(
        [ie[...], i1[...], i2[...], s10[...], s20[...], s11[...], s21[...]],
        axis=1)


def _tc_f(ue, u1, u2, ie, i1, i2, s10, s11, s20, s21,
          pa0, pa1, ru0, ru1, pe0, pe1, aux, ws):
    return pl.pallas_call(
        _tc_f_body,
        grid=(GRID,),
        in_specs=[_blk2(D)] * 10
        + [_blk3(80), _blk3(80), _blk3(D), _blk3(D), _blk3(D), _blk3(D),
           _blk2(8), _full((T, DT, DT))],
        out_specs=[_blk2(DSU), _blk2(DSI)],
        out_shape=[jax.ShapeDtypeStruct((U, DSU), jnp.float32),
                   jax.ShapeDtypeStruct((U, DSI), jnp.float32)],
    )(ue, u1, u2, ie, i1, i2, s10, s11, s20, s21,
      pa0, pa1, ru0, ru1, pe0, pe1, aux, ws)


def _pad_edges(src, dst):
    pad = EP - E
    srcp = jnp.concatenate([src.astype(jnp.int32), jnp.zeros((pad,), jnp.int32)])
    dstp = jnp.concatenate([dst.astype(jnp.int32),
                            jnp.full((pad,), DUMMY, jnp.int32)])
    return srcp.reshape(EP // CH, CH), dstp.reshape(EP // CH, CH)


def kernel(users, items, user_embedding, item_embedding, user_item_W,
           item_item_W, item_behavior_W_score, mgnn_weight, rel_user_idx,
           rel_item_idx, train_user_idx, train_item_idx, ii_src_idx,
           ii_dst_idx):
    f32 = jnp.float32
    onespad = jnp.concatenate(
        [jnp.ones((I, 1), f32), jnp.zeros((I, 15), f32)], axis=1)
    item_aug = jnp.concatenate([item_embedding, onespad], axis=1)
    user_aug = jnp.concatenate([user_embedding, onespad], axis=1)

    r0s, r0d = _pad_edges(rel_item_idx[0], rel_user_idx[0])
    r1s, r1d = _pad_edges(rel_item_idx[1], rel_user_idx[1])
    tts, ttd = _pad_edges(train_user_idx, train_item_idx)
    i0s, i0d = _pad_edges(ii_src_idx[0], ii_dst_idx[0])
    i1s, i1d = _pad_edges(ii_src_idx[1], ii_dst_idx[1])

    z64 = jnp.zeros((NA, 64), f32)
    z80 = jnp.zeros((NA, 80), f32)

    pa0, pa1, ptr, pi0, pi1 = _make_segsum((80, 80, 80, 80, 80))(
        item_aug, item_aug, user_aug, item_aug, item_aug,
        r0s, r1s, tts, i0s, i1s,
        r0d, r1d, ttd, i0d, i1d,
        z80)

    pa0s = pa0[:, :U, :]
    pa1s = pa1[:, :U, :]

    mg2 = mgnn_weight.reshape(1, T).astype(f32)
    u1, i1, s10, s11, c0, c1, aux = _tc_b(
        pa0s, pa1s, ptr[:, :U, :], pi0[:, :U, :], pi1[:, :U, :],
        mg2, user_item_W[0], item_item_W[:, 0])

    pt2, prI0, prS0, prI1, prS1, pii0, pii1 = _make_segsum((64,) * 7)(
        u1, i1, s10, i1, s11, s10, s11,
        tts, r0s, r0s, r1s, r1s, i0s, i1s,
        ttd, r0d, r0d, r1d, r1d, i0d, i1d,
        z64)

    u2, i2, s20, s21 = _tc_d(
        pt2[:, :U, :], prI0[:, :U, :], prI1[:, :U, :],
        pii0[:, :U, :], pii1[:, :U, :], aux,
        user_item_W[1], item_item_W[:, 1])

    pe0, pe1 = _make_segsum((64, 64))(s20, s21, r0s, r1s, r0d, r1d, z64)

    ubig, ibig = _tc_f(
        user_embedding, u1, u2, item_embedding, i1, i2, s10, s11, s20, s21,
        pa0s, pa1s, prS0[:, :U, :], prS1[:, :U, :],
        pe0[:, :U, :], pe1[:, :U, :], aux, item_behavior_W_score)

    items_p = jnp.concatenate(
        [items.astype(jnp.int32),
         jnp.zeros((NB, IPAD - NI), jnp.int32)], axis=1)
    scores_p, l2p = _make_score()(ubig, ibig, users.astype(jnp.int32), items_p)

    scores = scores_p[:, :NI]
    l2 = L2N * jnp.sum(l2p)
    return (scores, l2)
